# Initial kernel scaffold; baseline (speedup 1.0000x reference)
import jax
import jax.numpy as jnp
from jax.experimental import pallas as pl


def kernel(nodes, edges, timestamps, user_emb, item_emb, time_emb, user_neig, user_edge, user_time, user_mask, item_neig, item_edge, item_time, item_mask, Wq_0, Wk_0, Wv_0, Wm1_0, bm1_0, Wm2_0, bm2_0, Wq_1, Wk_1, Wv_1, Wm1_1, bm1_1, Wm2_1, bm2_1):
    raise NotImplementedError("write your pallas kernel here")


# speedup vs baseline: 40.9414x; 40.9414x over previous
import functools

import jax
import jax.numpy as jnp
import numpy as np
from jax import lax
from jax.experimental import pallas as pl
from jax.experimental.pallas import tpu as pltpu
from jax.experimental.pallas import tpu_sc as plsc

D = 64
H = 2
DH = D // H
TV = 20
K1 = 10
K2 = 20
NC = 2
NS = 16
NW = NC * NS
CH = 128


def _make_sc_gather(M, jobs):
    assert M % (8 * NW) == 0
    per_w = M // NW
    ch = min(CH, per_w)
    n_ch = per_w // ch
    assert per_w % ch == 0

    mesh = plsc.VectorSubcoreMesh(core_axis_name="c", subcore_axis_name="s")
    out_type = [jax.ShapeDtypeStruct((M, w), dt) for w, dt in jobs]
    scratch = []
    for w, dt in jobs:
        scratch += [pltpu.VMEM((ch,), jnp.int32),
                    pltpu.VMEM((ch, w), dt),
                    pltpu.SemaphoreType.DMA]

    @functools.partial(
        pl.kernel, mesh=mesh, out_type=out_type, scratch_types=scratch,
        compiler_params=pltpu.CompilerParams(use_tc_tiling_on_sc=False))
    def gather(*refs):
        nj = len(jobs)
        tabs = refs[:nj]
        idxs = refs[nj:2 * nj]
        outs = refs[2 * nj:3 * nj]
        scr = refs[3 * nj:]
        wid = lax.axis_index("s") * NC + lax.axis_index("c")
        base0 = wid * per_w

        def chunk(c, carry):
            base = pl.multiple_of(base0 + c * ch, 8)
            for j in range(nj):
                idx_v, row_v, sem = scr[3 * j], scr[3 * j + 1], scr[3 * j + 2]
                pltpu.sync_copy(idxs[j].at[pl.ds(base, ch)], idx_v)
                pltpu.async_copy(tabs[j].at[idx_v], row_v, sem).wait()
                pltpu.sync_copy(row_v, outs[j].at[pl.ds(base, ch)])
            return carry

        lax.fori_loop(0, n_ch, chunk, 0)

    return gather


def _attn_layer(nfeat, nemb, tms, msk, tnode, temb,
                Wq, Wk, Wv, Wm1, bm1, Wm2, bm2, k, nb):
    M = nfeat.shape[0]
    f32 = jnp.float32

    def body(nf_ref, ne_ref, tm_ref, mk_ref, tn_ref, te_ref,
             wq_ref, wk_ref, wv_ref, wm1_ref, b1_ref, wm2_ref, b2_ref, o_ref):
        pf = dict(preferred_element_type=f32)
        nf = nf_ref[...]
        te = te_ref[...]
        te0 = te[0:1, :]
        q = jnp.dot(nf, wq_ref[:D], **pf) + jnp.dot(te0, wq_ref[D:], **pf)
        tk = jnp.dot(te, wk_ref[D:], **pf)
        tv = jnp.dot(te, wv_ref[D:], **pf)
        delta = tn_ref[...] - tm_ref[...]
        bkt = jnp.mod(delta, TV)
        iot = lax.broadcasted_iota(jnp.int32, (nb, k, TV), 2)
        oh = (bkt[:, :, None] == iot).astype(f32).reshape(nb * k, TV)
        ne = ne_ref[...].reshape(nb * k, D)
        kk = (jnp.dot(ne, wk_ref[:D], **pf) + jnp.dot(oh, tk, **pf)
              ).reshape(nb, k, D)
        vv = (jnp.dot(ne, wv_ref[:D], **pf) + jnp.dot(oh, tv, **pf)
              ).reshape(nb, k, D)
        p = kk * q[:, None, :]
        lane = lax.broadcasted_iota(jnp.int32, (nb, k, D), 2)
        first = lane < DH
        s0 = jnp.sum(jnp.where(first, p, 0.0), axis=-1)
        s1 = jnp.sum(jnp.where(first, 0.0, p), axis=-1)
        scale = 1.0 / np.sqrt(DH)
        m = mk_ref[...] != 0
        s0 = jnp.where(m, -1e9, s0 * scale)
        s1 = jnp.where(m, -1e9, s1 * scale)

        def softmax(s):
            mx = jnp.max(s, axis=-1, keepdims=True)
            e = jnp.exp(s - mx)
            return e / jnp.sum(e, axis=-1, keepdims=True)

        a0 = softmax(s0)[:, :, None]
        a1 = softmax(s1)[:, :, None]
        att = jnp.where(first, a0, a1)
        out = jnp.sum(att * vv, axis=1)
        h = (jnp.dot(out, wm1_ref[:D], **pf) + jnp.dot(nf, wm1_ref[D:], **pf)
             + b1_ref[...])
        h = jnp.maximum(h, 0.0)
        o_ref[...] = jnp.dot(h, wm2_ref[...], **pf) + b2_ref[...]

    specs = [
        pl.BlockSpec((nb, D), lambda i: (i, 0)),
        pl.BlockSpec((nb, k, D), lambda i: (i, 0, 0)),
        pl.BlockSpec((nb, k), lambda i: (i, 0)),
        pl.BlockSpec((nb, k), lambda i: (i, 0)),
        pl.BlockSpec((nb, 1), lambda i: (i, 0)),
        pl.BlockSpec((TV, D), lambda i: (0, 0)),
        pl.BlockSpec((2 * D, D), lambda i: (0, 0)),
        pl.BlockSpec((2 * D, D), lambda i: (0, 0)),
        pl.BlockSpec((2 * D, D), lambda i: (0, 0)),
        pl.BlockSpec((2 * D, D), lambda i: (0, 0)),
        pl.BlockSpec((1, D), lambda i: (0, 0)),
        pl.BlockSpec((D, D), lambda i: (0, 0)),
        pl.BlockSpec((1, D), lambda i: (0, 0)),
    ]
    return pl.pallas_call(
        body, grid=(M // nb,), in_specs=specs,
        out_specs=pl.BlockSpec((nb, D), lambda i: (i, 0)),
        out_shape=jax.ShapeDtypeStruct((M, D), f32),
    )(nfeat, nemb, tms, msk, tnode, temb, Wq, Wk, Wv, Wm1,
      bm1.reshape(1, D), Wm2, bm2.reshape(1, D))


def kernel(nodes, edges, timestamps, user_emb, item_emb, time_emb,
           user_neig, user_edge, user_time, user_mask,
           item_neig, item_edge, item_time, item_mask,
           Wq_0, Wk_0, Wv_0, Wm1_0, bm1_0, Wm2_0, bm2_0,
           Wq_1, Wk_1, Wv_1, Wm1_1, bm1_1, Wm2_1, bm2_1):
    B = nodes.shape[0]
    i32 = jnp.int32
    f32 = jnp.float32
    nodes = nodes.astype(i32)
    edges = edges.astype(i32)
    timestamps = timestamps.astype(i32)
    user_emb = user_emb.astype(f32)
    item_emb = item_emb.astype(f32)
    time_emb = time_emb.astype(f32)

    utab = jnp.concatenate(
        [user_neig[:, -K2:].astype(i32), user_edge[:, -K2:].astype(i32),
         user_time[:, -K2:].astype(i32), user_mask[:, -K2:].astype(i32)],
        axis=1)
    itab = jnp.concatenate(
        [item_neig[:, -K1:].astype(i32), item_time[:, -K1:].astype(i32),
         item_mask[:, -K1:].astype(i32),
         jnp.zeros((item_neig.shape[0], 2), i32)],
        axis=1)

    urow, nfeat = _make_sc_gather(B, [(4 * K2, i32), (D, f32)])(
        utab, user_emb, edges, nodes)
    fa = urow[:, :K2].reshape(-1)
    fe = urow[:, K2:2 * K2].reshape(-1)
    tms2 = urow[:, 2 * K2:3 * K2]
    msk2 = urow[:, 3 * K2:]

    irow, ifeat = _make_sc_gather(B * K2, [(32, i32), (D, f32)])(
        itab, item_emb, fe, fa)
    adj1 = irow[:, :K1]
    tms1 = irow[:, K1:2 * K1]
    msk1 = irow[:, 2 * K1:3 * K1]

    (nemb0,) = _make_sc_gather(B * K2 * K1, [(D, f32)])(
        user_emb, adj1.reshape(-1))

    ft = tms2.reshape(B * K2, 1)
    item_out = _attn_layer(
        ifeat, nemb0.reshape(B * K2, K1, D), tms1, msk1, ft, time_emb,
        Wq_0, Wk_0, Wv_0, Wm1_0, bm1_0, Wm2_0, bm2_0, K1, 512)
    out = _attn_layer(
        nfeat, item_out.reshape(B, K2, D), tms2, msk2,
        timestamps.reshape(B, 1), time_emb,
        Wq_1, Wk_1, Wv_1, Wm1_1, bm1_1, Wm2_1, bm2_1, K2, 512)
    return out


# speedup vs baseline: 43.6646x; 1.0665x over previous
import functools

import jax
import jax.numpy as jnp
import numpy as np
from jax import lax
from jax.experimental import pallas as pl
from jax.experimental.pallas import tpu as pltpu
from jax.experimental.pallas import tpu_sc as plsc

D = 64
H = 2
DH = D // H
TV = 20
K1 = 10
K2 = 20
NC = 2
NS = 16
NW = NC * NS
CH = 128


def _make_sc_gather(M, jobs):
    assert M % (8 * NW) == 0
    per_w = M // NW
    ch = min(CH, per_w)
    n_ch = per_w // ch
    assert per_w % ch == 0

    mesh = plsc.VectorSubcoreMesh(core_axis_name="c", subcore_axis_name="s")
    out_type = [jax.ShapeDtypeStruct((M, w), dt) for w, dt in jobs]
    scratch = []
    for w, dt in jobs:
        scratch += [pltpu.VMEM((ch,), jnp.int32),
                    pltpu.VMEM((ch, w), dt),
                    pltpu.SemaphoreType.DMA]

    @functools.partial(
        pl.kernel, mesh=mesh, out_type=out_type, scratch_types=scratch,
        compiler_params=pltpu.CompilerParams(use_tc_tiling_on_sc=False))
    def gather(*refs):
        nj = len(jobs)
        tabs = refs[:nj]
        idxs = refs[nj:2 * nj]
        outs = refs[2 * nj:3 * nj]
        scr = refs[3 * nj:]
        wid = lax.axis_index("s") * NC + lax.axis_index("c")
        base0 = wid * per_w

        def chunk(c, carry):
            base = pl.multiple_of(base0 + c * ch, 8)
            for j in range(nj):
                idx_v, row_v, sem = scr[3 * j], scr[3 * j + 1], scr[3 * j + 2]
                pltpu.sync_copy(idxs[j].at[pl.ds(base, ch)], idx_v)
                pltpu.async_copy(tabs[j].at[idx_v], row_v, sem).wait()
                pltpu.sync_copy(row_v, outs[j].at[pl.ds(base, ch)])
            return carry

        lax.fori_loop(0, n_ch, chunk, 0)

    return gather


def _attn_layer(nfeat, nemb_t, tms_t, msk_t, tnode, temb,
                Wq, Wk, Wv, Wm1, bm1, Wm2, bm2, k, nb):
    M = nfeat.shape[0]
    f32 = jnp.float32

    def body(nf_ref, ne_ref, tm_ref, mk_ref, tn_ref, te_ref,
             wq_ref, wk_ref, wv_ref, wm1_ref, b1_ref, wm2_ref, b2_ref, o_ref):
        pf = dict(preferred_element_type=f32)
        nf = nf_ref[...]
        te = te_ref[...]
        te0 = te[0:1, :]
        q = jnp.dot(nf, wq_ref[:D], **pf) + jnp.dot(te0, wq_ref[D:], **pf)
        tk = jnp.dot(te, wk_ref[D:], **pf)
        tv = jnp.dot(te, wv_ref[D:], **pf)
        delta = (tn_ref[...] - tm_ref[...]).astype(jnp.int32)
        bkt = jnp.mod(delta, TV)
        iot = lax.broadcasted_iota(jnp.int32, (k, nb, TV), 2)
        oh = (bkt == iot).astype(f32).reshape(k * nb, TV)
        ne = ne_ref[...].reshape(k * nb, D)
        kk = jnp.dot(ne, wk_ref[:D], **pf) + jnp.dot(oh, tk, **pf)
        vv = (jnp.dot(ne, wv_ref[:D], **pf) + jnp.dot(oh, tv, **pf)
              ).reshape(k, nb, D)
        qb = jnp.broadcast_to(q[None], (k, nb, D)).reshape(k * nb, D)
        rr = lax.broadcasted_iota(jnp.int32, (D, D), 0)
        cc = lax.broadcasted_iota(jnp.int32, (D, D), 1)
        sel = ((rr < DH) == (cc < DH)).astype(f32)
        s = jnp.dot(kk * qb, sel, **pf).reshape(k, nb, D) * (1.0 / np.sqrt(DH))
        s = jnp.where(mk_ref[...] > 0.5, -1e9, s)
        mx = s[0]
        for j in range(1, k):
            mx = jnp.maximum(mx, s[j])
        e = jnp.exp(s - mx[None])
        den = e[0]
        for j in range(1, k):
            den = den + e[j]
        a = e * (1.0 / den)[None]
        w = a * vv
        out = w[0]
        for j in range(1, k):
            out = out + w[j]
        h = (jnp.dot(out, wm1_ref[:D], **pf) + jnp.dot(nf, wm1_ref[D:], **pf)
             + b1_ref[...])
        h = jnp.maximum(h, 0.0)
        o_ref[...] = jnp.dot(h, wm2_ref[...], **pf) + b2_ref[...]

    specs = [
        pl.BlockSpec((nb, D), lambda i: (i, 0)),
        pl.BlockSpec((k, nb, D), lambda i: (0, i, 0)),
        pl.BlockSpec((k, nb, 1), lambda i: (0, i, 0)),
        pl.BlockSpec((k, nb, 1), lambda i: (0, i, 0)),
        pl.BlockSpec((1, nb, 1), lambda i: (0, i, 0)),
        pl.BlockSpec((TV, D), lambda i: (0, 0)),
        pl.BlockSpec((2 * D, D), lambda i: (0, 0)),
        pl.BlockSpec((2 * D, D), lambda i: (0, 0)),
        pl.BlockSpec((2 * D, D), lambda i: (0, 0)),
        pl.BlockSpec((2 * D, D), lambda i: (0, 0)),
        pl.BlockSpec((1, D), lambda i: (0, 0)),
        pl.BlockSpec((D, D), lambda i: (0, 0)),
        pl.BlockSpec((1, D), lambda i: (0, 0)),
    ]
    return pl.pallas_call(
        body, grid=(M // nb,), in_specs=specs,
        out_specs=pl.BlockSpec((nb, D), lambda i: (i, 0)),
        out_shape=jax.ShapeDtypeStruct((M, D), f32),
    )(nfeat, nemb_t, tms_t, msk_t, tnode, temb, Wq, Wk, Wv, Wm1,
      bm1.reshape(1, D), Wm2, bm2.reshape(1, D))


def kernel(nodes, edges, timestamps, user_emb, item_emb, time_emb,
           user_neig, user_edge, user_time, user_mask,
           item_neig, item_edge, item_time, item_mask,
           Wq_0, Wk_0, Wv_0, Wm1_0, bm1_0, Wm2_0, bm2_0,
           Wq_1, Wk_1, Wv_1, Wm1_1, bm1_1, Wm2_1, bm2_1):
    B = nodes.shape[0]
    i32 = jnp.int32
    f32 = jnp.float32
    nodes = nodes.astype(i32)
    edges = edges.astype(i32)
    timestamps = timestamps.astype(i32)
    user_emb = user_emb.astype(f32)
    item_emb = item_emb.astype(f32)
    time_emb = time_emb.astype(f32)

    utab = jnp.concatenate(
        [user_neig[:, -K2:].astype(i32), user_edge[:, -K2:].astype(i32),
         user_time[:, -K2:].astype(i32), user_mask[:, -K2:].astype(i32)],
        axis=1)
    itab = jnp.concatenate(
        [item_neig[:, -K1:].astype(i32), item_time[:, -K1:].astype(i32),
         item_mask[:, -K1:].astype(i32),
         jnp.zeros((item_neig.shape[0], 2), i32)],
        axis=1)

    urow, nfeat = _make_sc_gather(B, [(4 * K2, i32), (D, f32)])(
        utab, user_emb, edges, nodes)
    fa_t = urow[:, :K2].T.reshape(-1)
    fe_t = urow[:, K2:2 * K2].T.reshape(-1)
    tms2 = urow[:, 2 * K2:3 * K2]
    msk2 = urow[:, 3 * K2:]

    M1 = B * K2
    irow, ifeat = _make_sc_gather(M1, [(32, i32), (D, f32)])(
        itab, item_emb, fe_t, fa_t)
    adj1 = irow[:, :K1]
    tms1_t = irow[:, K1:2 * K1].T.astype(f32).reshape(K1, M1, 1)
    msk1_t = irow[:, 2 * K1:3 * K1].T.astype(f32).reshape(K1, M1, 1)

    (nemb0,) = _make_sc_gather(M1 * K1, [(D, f32)])(
        user_emb, adj1.T.reshape(-1))

    ft_t = tms2.T.reshape(1, M1, 1).astype(f32)
    item_out = _attn_layer(
        ifeat, nemb0.reshape(K1, M1, D), tms1_t, msk1_t, ft_t, time_emb,
        Wq_0, Wk_0, Wv_0, Wm1_0, bm1_0, Wm2_0, bm2_0, K1, 512)
    out = _attn_layer(
        nfeat, item_out.reshape(K2, B, D),
        tms2.T.astype(f32).reshape(K2, B, 1),
        msk2.T.astype(f32).reshape(K2, B, 1),
        timestamps.reshape(1, B, 1).astype(f32), time_emb,
        Wq_1, Wk_1, Wv_1, Wm1_1, bm1_1, Wm2_1, bm2_1, K2, 512)
    return out
